# baseline (speedup 1.0000x reference)
import dataclasses
import functools

import jax
import jax.numpy as jnp
from jax import lax
from jax.experimental import pallas as pl
from jax.experimental.pallas import tpu as pltpu
from jax.experimental.pallas import tpu_sc as plsc

NC = 2
NS = 16
NW = NC * NS
KW = 128


def _sc_mesh():
    return plsc.VectorSubcoreMesh(
        core_axis_name="c", subcore_axis_name="s", num_cores=NC, num_subcores=NS
    )


def _deg_kernel(n_tot, nwin):
    rpt = n_tot // NS

    @functools.partial(
        pl.kernel,
        out_type=jax.ShapeDtypeStruct((NC, n_tot), jnp.float32),
        mesh=_sc_mesh(),
        scratch_types=[
            pltpu.VMEM((nwin, KW), jnp.int32),
            pltpu.VMEM((KW,), jnp.float32),
            pltpu.VMEM_SHARED((n_tot,), jnp.float32),
            pltpu.SemaphoreType.DMA,
        ],
    )
    def k(dst_hbm, zeros_hbm, ones_hbm, out_hbm, dstv, onesv, acc, ss):
        c = lax.axis_index("c")
        s = lax.axis_index("s")
        wid = c * NS + s
        pltpu.sync_copy(dst_hbm.at[wid], dstv)
        pltpu.sync_copy(ones_hbm, onesv)
        pltpu.sync_copy(zeros_hbm.at[pl.ds(s * rpt, rpt)],
                        acc.at[pl.ds(s * rpt, rpt)])
        plsc.subcore_barrier()

        @pl.loop(0, nwin)
        def _(w):
            pltpu.async_copy(onesv, acc.at[dstv.at[w]], ss, add=True)

        @pl.loop(0, nwin)
        def _(w):
            pltpu.make_async_copy(onesv, acc.at[dstv.at[w]], ss).wait()

        plsc.subcore_barrier()
        pltpu.sync_copy(acc.at[pl.ds(s * rpt, rpt)],
                        out_hbm.at[c].at[pl.ds(s * rpt, rpt)])

    return k


GB = 16


def _agg_kernel(n_tot, h, nwin):
    rpt = n_tot // NS
    assert nwin % GB == 0
    nchunk = nwin // GB

    @functools.partial(
        pl.kernel,
        out_type=jax.ShapeDtypeStruct((NC, n_tot, h), jnp.float32),
        mesh=_sc_mesh(),
        scratch_types=[
            pltpu.VMEM((GB, KW), jnp.int32),
            pltpu.VMEM((GB, KW), jnp.int32),
            pltpu.VMEM((KW // 2, h), jnp.float32),
            pltpu.VMEM((KW // 2, h), jnp.float32),
            pltpu.VMEM((KW // 2, h), jnp.float32),
            pltpu.VMEM((KW // 2, h), jnp.float32),
            pltpu.VMEM_SHARED((n_tot, h), jnp.float32),
            [pltpu.SemaphoreType.DMA] * 4,
            [pltpu.SemaphoreType.DMA] * 4,
        ],
    )
    def k(g_hbm, src_hbm, dst_hbm, zeros_hbm, out_hbm,
          srcv, dstv, b0, b1, b2, b3, acc, gss, sss):
        c = lax.axis_index("c")
        s = lax.axis_index("s")
        wid = c * NS + s
        bb = (b0, b1, b2, b3)
        hw = KW // 2
        nh = 2 * GB
        pltpu.sync_copy(zeros_hbm.at[pl.ds(s * rpt, rpt)],
                        acc.at[pl.ds(s * rpt, rpt)])
        plsc.subcore_barrier()

        @pl.loop(0, nchunk)
        def _(ci):
            pltpu.sync_copy(src_hbm.at[wid].at[pl.ds(ci * GB, GB)], srcv)
            pltpu.sync_copy(dst_hbm.at[wid].at[pl.ds(ci * GB, GB)], dstv)

            def sidx(ref, t):
                return ref.at[t // 2].at[pl.ds((t % 2) * hw, hw)]

            hg = [None, None, None, None]
            hs = [None, None, None, None]
            for t in range(nh):
                p = t % 4
                if hs[p] is not None:
                    hs[p].wait()
                hg[p] = pltpu.async_copy(g_hbm.at[sidx(srcv, t)], bb[p],
                                         gss[p], priority=1)
                if t >= 1:
                    q = (t - 1) % 4
                    hg[q].wait()
                    hs[q] = pltpu.async_copy(bb[q], acc.at[sidx(dstv, t - 1)],
                                             sss[q], add=True)
            p = (nh - 1) % 4
            hg[p].wait()
            hs[p] = pltpu.async_copy(bb[p], acc.at[sidx(dstv, nh - 1)],
                                     sss[p], add=True)
            for p in range(4):
                if hs[p] is not None:
                    hs[p].wait()

        plsc.subcore_barrier()
        pltpu.sync_copy(acc.at[pl.ds(s * rpt, rpt)],
                        out_hbm.at[c].at[pl.ds(s * rpt, rpt)])

    return k


def _agg1_kernel(n_tot, nwin):
    rpt = n_tot // NS

    cp = pltpu.CompilerParams()
    if "needs_layout_passes" in pltpu.CompilerParams.__dataclass_fields__:
        cp = dataclasses.replace(cp, needs_layout_passes=False)

    @functools.partial(
        pl.kernel,
        out_type=jax.ShapeDtypeStruct((NC, n_tot), jnp.float32),
        mesh=_sc_mesh(),
        compiler_params=cp,
        scratch_types=[
            pltpu.VMEM((n_tot,), jnp.float32),
            pltpu.VMEM((nwin, KW), jnp.int32),
            pltpu.VMEM((nwin, KW), jnp.int32),
            pltpu.VMEM((nwin, KW), jnp.float32),
            pltpu.VMEM_SHARED((n_tot,), jnp.float32),
            pltpu.SemaphoreType.DMA,
        ],
    )
    def k(z_hbm, src_hbm, dst_hbm, zeros_hbm, out_hbm, zv, srcv, dstv, vals,
          acc, ss):
        c = lax.axis_index("c")
        s = lax.axis_index("s")
        wid = c * NS + s
        pltpu.sync_copy(z_hbm, zv)
        pltpu.sync_copy(src_hbm.at[wid], srcv)
        pltpu.sync_copy(dst_hbm.at[wid], dstv)
        pltpu.sync_copy(zeros_hbm.at[pl.ds(s * rpt, rpt)],
                        acc.at[pl.ds(s * rpt, rpt)])
        plsc.subcore_barrier()

        @pl.loop(0, nwin)
        def _(w):
            for g in range(KW // 16):
                idx = srcv[w, pl.ds(16 * g, 16)]
                vals[w, pl.ds(16 * g, 16)] = plsc.load_gather(zv, [idx])

        @pl.loop(0, nwin)
        def _(w):
            pltpu.async_copy(vals.at[w], acc.at[dstv.at[w]], ss, add=True)

        @pl.loop(0, nwin)
        def _(w):
            pltpu.make_async_copy(vals.at[w], acc.at[dstv.at[w]], ss).wait()

        plsc.subcore_barrier()
        pltpu.sync_copy(acc.at[pl.ds(s * rpt, rpt)],
                        out_hbm.at[c].at[pl.ds(s * rpt, rpt)])

    return k


def _matmul(x, w, bn):
    n, d = x.shape
    h = w.shape[1]

    def body(x_ref, w_ref, o_ref):
        o_ref[...] = jnp.dot(x_ref[...], w_ref[...],
                             precision=lax.Precision.HIGHEST,
                             preferred_element_type=jnp.float32)

    return pl.pallas_call(
        body,
        grid=(n // bn,),
        in_specs=[pl.BlockSpec((bn, d), lambda i: (i, 0)),
                  pl.BlockSpec((d, h), lambda i: (0, 0))],
        out_specs=pl.BlockSpec((bn, h), lambda i: (i, 0)),
        out_shape=jax.ShapeDtypeStruct((n, h), jnp.float32),
    )(x, w)


def _scale_rows(h_ext, deg0, deg1, n_real, bn):
    n_tot, hdim = h_ext.shape

    def body(h_ref, d0_ref, d1_ref, g_ref, dinv_ref):
        i = pl.program_id(0)
        rid = i * bn + lax.broadcasted_iota(jnp.int32, (bn,), 0)
        dinv = lax.rsqrt(1.0 + d0_ref[...] + d1_ref[...])
        dinv = jnp.where(rid < n_real, dinv, 0.0)
        dinv_ref[...] = dinv
        g_ref[...] = h_ref[...] * dinv[:, None]

    return pl.pallas_call(
        body,
        grid=(n_tot // bn,),
        in_specs=[pl.BlockSpec((bn, hdim), lambda i: (i, 0)),
                  pl.BlockSpec((bn,), lambda i: (i,)),
                  pl.BlockSpec((bn,), lambda i: (i,))],
        out_specs=[pl.BlockSpec((bn, hdim), lambda i: (i, 0)),
                   pl.BlockSpec((bn,), lambda i: (i,))],
        out_shape=[jax.ShapeDtypeStruct((n_tot, hdim), jnp.float32),
                   jax.ShapeDtypeStruct((n_tot,), jnp.float32)],
    )(h_ext, deg0, deg1)


def _layer2_z(s0, s1, g, dinv, b1, w2row, bn):
    n_tot, hdim = g.shape

    def body(s0_ref, s1_ref, g_ref, dinv_ref, b1_ref, w2_ref, o_ref):
        dinv = dinv_ref[...]
        t = dinv[:, None] * (s0_ref[...] + s1_ref[...] + g_ref[...])
        t = jnp.maximum(t + b1_ref[...], 0.0)
        o_ref[...] = jnp.sum(t * w2_ref[...], axis=1) * dinv

    return pl.pallas_call(
        body,
        grid=(n_tot // bn,),
        in_specs=[pl.BlockSpec((bn, hdim), lambda i: (i, 0)),
                  pl.BlockSpec((bn, hdim), lambda i: (i, 0)),
                  pl.BlockSpec((bn, hdim), lambda i: (i, 0)),
                  pl.BlockSpec((bn,), lambda i: (i,)),
                  pl.BlockSpec((1, hdim), lambda i: (0, 0)),
                  pl.BlockSpec((1, hdim), lambda i: (0, 0))],
        out_specs=pl.BlockSpec((bn,), lambda i: (i,)),
        out_shape=jax.ShapeDtypeStruct((n_tot,), jnp.float32),
    )(s0, s1, g, dinv, b1, w2row)


def _final(s0, s1, zp, dinv, b2, bn):
    n = s0.shape[0]

    def body(s0_ref, s1_ref, z_ref, dinv_ref, b2_ref, o_ref):
        o_ref[...] = (dinv_ref[...] * (s0_ref[...] + s1_ref[...] + z_ref[...])
                      + b2_ref[...])

    return pl.pallas_call(
        body,
        grid=(n // bn,),
        in_specs=[pl.BlockSpec((bn,), lambda i: (i,)),
                  pl.BlockSpec((bn,), lambda i: (i,)),
                  pl.BlockSpec((bn,), lambda i: (i,)),
                  pl.BlockSpec((bn,), lambda i: (i,)),
                  pl.BlockSpec((1,), lambda i: (0,))],
        out_specs=pl.BlockSpec((bn,), lambda i: (i,)),
        out_shape=jax.ShapeDtypeStruct((n,), jnp.float32),
    )(s0, s1, zp, dinv, b2)


def kernel(x, edge_index, W1, b1, W2, b2):
    n, d = x.shape
    h = W1.shape[1]
    e = edge_index.shape[1]

    pad_rows = 240
    n_tot = n + pad_rows
    nwin = -(-e // (NW * KW))
    nwin = -(-nwin // GB) * GB
    e_pad = NW * nwin * KW

    npad = e_pad - e
    pad_idx = (n + jnp.arange(npad, dtype=jnp.int32) % pad_rows)
    src3 = jnp.concatenate([edge_index[0].astype(jnp.int32), pad_idx])
    dst3 = jnp.concatenate([edge_index[1].astype(jnp.int32), pad_idx])
    src3 = src3.reshape(NW, nwin, KW)
    dst3 = dst3.reshape(NW, nwin, KW)

    zeros1 = jnp.zeros((n_tot,), jnp.float32)
    zeros2 = jnp.zeros((n_tot, h), jnp.float32)
    ones = jnp.ones((KW,), jnp.float32)

    x_ext = jnp.pad(x, ((0, pad_rows), (0, 0)))

    degp = _deg_kernel(n_tot, nwin)(dst3, zeros1, ones)
    h_ext = _matmul(x_ext, W1, bn=1280)

    g_ext, dinv = _scale_rows(h_ext, degp[0], degp[1], n, bn=1024)

    sp = _agg_kernel(n_tot, h, nwin)(g_ext, src3, dst3, zeros2)

    zp = _layer2_z(sp[0], sp[1], g_ext, dinv, b1.reshape(1, h),
                   W2.reshape(1, h), bn=1024)

    s2p = _agg1_kernel(n_tot, nwin)(zp, src3, dst3, zeros1)

    return _final(s2p[0], s2p[1], zp, dinv, b2, bn=1024)[:n]


# speedup vs baseline: 1.1014x; 1.1014x over previous
import dataclasses
import functools

import jax
import jax.numpy as jnp
from jax import lax
from jax.experimental import pallas as pl
from jax.experimental.pallas import tpu as pltpu
from jax.experimental.pallas import tpu_sc as plsc

NC = 2
NS = 16
NW = NC * NS
KW = 128


def _sc_mesh():
    return plsc.VectorSubcoreMesh(
        core_axis_name="c", subcore_axis_name="s", num_cores=NC, num_subcores=NS
    )


def _deg_kernel(n_tot, nwin):
    rpt = n_tot // NS

    @functools.partial(
        pl.kernel,
        out_type=jax.ShapeDtypeStruct((NC, n_tot), jnp.float32),
        mesh=_sc_mesh(),
        scratch_types=[
            pltpu.VMEM((nwin, KW), jnp.int32),
            pltpu.VMEM((KW,), jnp.float32),
            pltpu.VMEM_SHARED((n_tot,), jnp.float32),
            pltpu.SemaphoreType.DMA,
        ],
    )
    def k(dst_hbm, zeros_hbm, ones_hbm, out_hbm, dstv, onesv, acc, ss):
        c = lax.axis_index("c")
        s = lax.axis_index("s")
        wid = c * NS + s
        pltpu.sync_copy(dst_hbm.at[wid], dstv)
        pltpu.sync_copy(ones_hbm, onesv)
        pltpu.sync_copy(zeros_hbm.at[pl.ds(s * rpt, rpt)],
                        acc.at[pl.ds(s * rpt, rpt)])
        plsc.subcore_barrier()

        @pl.loop(0, nwin)
        def _(w):
            pltpu.async_copy(onesv, acc.at[dstv.at[w]], ss, add=True)

        @pl.loop(0, nwin)
        def _(w):
            pltpu.make_async_copy(onesv, acc.at[dstv.at[w]], ss).wait()

        plsc.subcore_barrier()
        pltpu.sync_copy(acc.at[pl.ds(s * rpt, rpt)],
                        out_hbm.at[c].at[pl.ds(s * rpt, rpt)])

    return k


GB = 40


def _agg_kernel(n_tot, h, nwin):
    rpt = n_tot // NS
    assert nwin % GB == 0
    nchunk = nwin // GB

    @functools.partial(
        pl.kernel,
        out_type=jax.ShapeDtypeStruct((NC, n_tot, h), jnp.float32),
        mesh=_sc_mesh(),
        scratch_types=[
            pltpu.VMEM((GB, KW), jnp.int32),
            pltpu.VMEM((GB, KW), jnp.int32),
            pltpu.VMEM((KW, h), jnp.float32),
            pltpu.VMEM((KW, h), jnp.float32),
            pltpu.VMEM_SHARED((n_tot, h), jnp.float32),
            pltpu.SemaphoreType.DMA,
            pltpu.SemaphoreType.DMA,
            pltpu.SemaphoreType.DMA,
            pltpu.SemaphoreType.DMA,
        ],
    )
    def k(g_hbm, src_hbm, dst_hbm, zeros_hbm, out_hbm,
          srcv, dstv, b0, b1, acc, gs0, gs1, ss0, ss1):
        c = lax.axis_index("c")
        s = lax.axis_index("s")
        wid = c * NS + s
        bb = (b0, b1)
        gss = (gs0, gs1)
        sss = (ss0, ss1)
        pltpu.sync_copy(zeros_hbm.at[pl.ds(s * rpt, rpt)],
                        acc.at[pl.ds(s * rpt, rpt)])
        plsc.subcore_barrier()

        @pl.loop(0, nchunk)
        def _(ci):
            pltpu.sync_copy(src_hbm.at[wid].at[pl.ds(ci * GB, GB)], srcv)
            pltpu.sync_copy(dst_hbm.at[wid].at[pl.ds(ci * GB, GB)], dstv)
            hg = [None, None]
            hs = [None, None]
            for j in range(GB):
                p = j & 1
                if hs[p] is not None:
                    hs[p].wait()
                hg[p] = pltpu.async_copy(g_hbm.at[srcv.at[j]], bb[p], gss[p],
                                         priority=1)
                if j >= 1:
                    q = 1 - p
                    hg[q].wait()
                    hs[q] = pltpu.async_copy(bb[q], acc.at[dstv.at[j - 1]],
                                             sss[q], add=True)
            p = (GB - 1) & 1
            hg[p].wait()
            hs[p] = pltpu.async_copy(bb[p], acc.at[dstv.at[GB - 1]],
                                     sss[p], add=True)
            hs[0].wait()
            hs[1].wait()

        plsc.subcore_barrier()
        pltpu.sync_copy(acc.at[pl.ds(s * rpt, rpt)],
                        out_hbm.at[c].at[pl.ds(s * rpt, rpt)])

    return k


def _agg1_kernel(n_tot, nwin):
    rpt = n_tot // NS

    cp = pltpu.CompilerParams()
    if "needs_layout_passes" in pltpu.CompilerParams.__dataclass_fields__:
        cp = dataclasses.replace(cp, needs_layout_passes=False)

    @functools.partial(
        pl.kernel,
        out_type=jax.ShapeDtypeStruct((NC, n_tot), jnp.float32),
        mesh=_sc_mesh(),
        compiler_params=cp,
        scratch_types=[
            pltpu.VMEM((n_tot,), jnp.float32),
            pltpu.VMEM((nwin, KW), jnp.int32),
            pltpu.VMEM((nwin, KW), jnp.int32),
            pltpu.VMEM((nwin, KW), jnp.float32),
            pltpu.VMEM_SHARED((n_tot,), jnp.float32),
            pltpu.SemaphoreType.DMA,
        ],
    )
    def k(z_hbm, src_hbm, dst_hbm, zeros_hbm, out_hbm, zv, srcv, dstv, vals,
          acc, ss):
        c = lax.axis_index("c")
        s = lax.axis_index("s")
        wid = c * NS + s
        pltpu.sync_copy(z_hbm, zv)
        pltpu.sync_copy(src_hbm.at[wid], srcv)
        pltpu.sync_copy(dst_hbm.at[wid], dstv)
        pltpu.sync_copy(zeros_hbm.at[pl.ds(s * rpt, rpt)],
                        acc.at[pl.ds(s * rpt, rpt)])
        plsc.subcore_barrier()

        @pl.loop(0, nwin)
        def _(w):
            for g in range(KW // 16):
                idx = srcv[w, pl.ds(16 * g, 16)]
                vals[w, pl.ds(16 * g, 16)] = plsc.load_gather(zv, [idx])

        @pl.loop(0, nwin)
        def _(w):
            pltpu.async_copy(vals.at[w], acc.at[dstv.at[w]], ss, add=True)

        @pl.loop(0, nwin)
        def _(w):
            pltpu.make_async_copy(vals.at[w], acc.at[dstv.at[w]], ss).wait()

        plsc.subcore_barrier()
        pltpu.sync_copy(acc.at[pl.ds(s * rpt, rpt)],
                        out_hbm.at[c].at[pl.ds(s * rpt, rpt)])

    return k


def _scaled_matmul(x, w, deg0, deg1, n_real, bn):
    n, d = x.shape
    h = w.shape[1]

    def body(x_ref, w_ref, d0_ref, d1_ref, g_ref, dinv_ref):
        i = pl.program_id(0)
        rid = i * bn + lax.broadcasted_iota(jnp.int32, (bn,), 0)
        dinv = lax.rsqrt(1.0 + d0_ref[...] + d1_ref[...])
        dinv = jnp.where(rid < n_real, dinv, 0.0)
        dinv_ref[...] = dinv
        g_ref[...] = jnp.dot(x_ref[...] * dinv[:, None], w_ref[...],
                             precision=lax.Precision.HIGHEST,
                             preferred_element_type=jnp.float32)

    return pl.pallas_call(
        body,
        grid=(n // bn,),
        in_specs=[pl.BlockSpec((bn, d), lambda i: (i, 0)),
                  pl.BlockSpec((d, h), lambda i: (0, 0)),
                  pl.BlockSpec((bn,), lambda i: (i,)),
                  pl.BlockSpec((bn,), lambda i: (i,))],
        out_specs=[pl.BlockSpec((bn, h), lambda i: (i, 0)),
                   pl.BlockSpec((bn,), lambda i: (i,))],
        out_shape=[jax.ShapeDtypeStruct((n, h), jnp.float32),
                   jax.ShapeDtypeStruct((n,), jnp.float32)],
    )(x, w, deg0, deg1)


def _layer2_z(s0, s1, g, dinv, b1, w2row, bn):
    n_tot, hdim = g.shape

    def body(s0_ref, s1_ref, g_ref, dinv_ref, b1_ref, w2_ref, o_ref):
        dinv = dinv_ref[...]
        t = dinv[:, None] * (s0_ref[...] + s1_ref[...] + g_ref[...])
        t = jnp.maximum(t + b1_ref[...], 0.0)
        o_ref[...] = jnp.sum(t * w2_ref[...], axis=1) * dinv

    return pl.pallas_call(
        body,
        grid=(n_tot // bn,),
        in_specs=[pl.BlockSpec((bn, hdim), lambda i: (i, 0)),
                  pl.BlockSpec((bn, hdim), lambda i: (i, 0)),
                  pl.BlockSpec((bn, hdim), lambda i: (i, 0)),
                  pl.BlockSpec((bn,), lambda i: (i,)),
                  pl.BlockSpec((1, hdim), lambda i: (0, 0)),
                  pl.BlockSpec((1, hdim), lambda i: (0, 0))],
        out_specs=pl.BlockSpec((bn,), lambda i: (i,)),
        out_shape=jax.ShapeDtypeStruct((n_tot,), jnp.float32),
    )(s0, s1, g, dinv, b1, w2row)


def _final(s0, s1, zp, dinv, b2, bn):
    n = s0.shape[0]

    def body(s0_ref, s1_ref, z_ref, dinv_ref, b2_ref, o_ref):
        o_ref[...] = (dinv_ref[...] * (s0_ref[...] + s1_ref[...] + z_ref[...])
                      + b2_ref[...])

    return pl.pallas_call(
        body,
        grid=(n // bn,),
        in_specs=[pl.BlockSpec((bn,), lambda i: (i,)),
                  pl.BlockSpec((bn,), lambda i: (i,)),
                  pl.BlockSpec((bn,), lambda i: (i,)),
                  pl.BlockSpec((bn,), lambda i: (i,)),
                  pl.BlockSpec((1,), lambda i: (0,))],
        out_specs=pl.BlockSpec((bn,), lambda i: (i,)),
        out_shape=jax.ShapeDtypeStruct((n,), jnp.float32),
    )(s0, s1, zp, dinv, b2)


def kernel(x, edge_index, W1, b1, W2, b2):
    n, d = x.shape
    h = W1.shape[1]
    e = edge_index.shape[1]

    pad_rows = 240
    n_tot = n + pad_rows
    nwin = -(-e // (NW * KW))
    nwin = -(-nwin // GB) * GB
    e_pad = NW * nwin * KW

    npad = e_pad - e
    pad_idx = (n + jnp.arange(npad, dtype=jnp.int32) % pad_rows)
    src3 = jnp.concatenate([edge_index[0].astype(jnp.int32), pad_idx])
    dst3 = jnp.concatenate([edge_index[1].astype(jnp.int32), pad_idx])
    src3 = src3.reshape(NW, nwin, KW)
    dst3 = dst3.reshape(NW, nwin, KW)

    zeros1 = jnp.zeros((n_tot,), jnp.float32)
    zeros2 = jnp.zeros((n_tot, h), jnp.float32)
    ones = jnp.ones((KW,), jnp.float32)

    x_ext = jnp.pad(x, ((0, pad_rows), (0, 0)))

    degp = _deg_kernel(n_tot, nwin)(dst3, zeros1, ones)
    g_ext, dinv = _scaled_matmul(x_ext, W1, degp[0], degp[1], n, bn=1024)

    sp = _agg_kernel(n_tot, h, nwin)(g_ext, src3, dst3, zeros2)

    zp = _layer2_z(sp[0], sp[1], g_ext, dinv, b1.reshape(1, h),
                   W2.reshape(1, h), bn=1024)

    s2p = _agg1_kernel(n_tot, nwin)(zp, src3, dst3, zeros1)

    return _final(s2p[0], s2p[1], zp, dinv, b2, bn=1024)[:n]


# speedup vs baseline: 1.1016x; 1.0002x over previous
import dataclasses
import functools

import jax
import jax.numpy as jnp
from jax import lax
from jax.experimental import pallas as pl
from jax.experimental.pallas import tpu as pltpu
from jax.experimental.pallas import tpu_sc as plsc

NC = 2
NS = 16
NW = NC * NS
KW = 128


def _sc_mesh():
    return plsc.VectorSubcoreMesh(
        core_axis_name="c", subcore_axis_name="s", num_cores=NC, num_subcores=NS
    )


def _deg_kernel(n_tot, nwin):
    rpt = n_tot // NS

    @functools.partial(
        pl.kernel,
        out_type=jax.ShapeDtypeStruct((NC, n_tot), jnp.float32),
        mesh=_sc_mesh(),
        scratch_types=[
            pltpu.VMEM((nwin, KW), jnp.int32),
            pltpu.VMEM((KW,), jnp.float32),
            pltpu.VMEM_SHARED((n_tot,), jnp.float32),
            pltpu.SemaphoreType.DMA,
        ],
    )
    def k(dst_hbm, zeros_hbm, ones_hbm, out_hbm, dstv, onesv, acc, ss):
        c = lax.axis_index("c")
        s = lax.axis_index("s")
        wid = c * NS + s
        pltpu.sync_copy(dst_hbm.at[wid], dstv)
        pltpu.sync_copy(ones_hbm, onesv)
        pltpu.sync_copy(zeros_hbm.at[pl.ds(s * rpt, rpt)],
                        acc.at[pl.ds(s * rpt, rpt)])
        plsc.subcore_barrier()

        @pl.loop(0, nwin)
        def _(w):
            pltpu.async_copy(onesv, acc.at[dstv.at[w]], ss, add=True)

        @pl.loop(0, nwin)
        def _(w):
            pltpu.make_async_copy(onesv, acc.at[dstv.at[w]], ss).wait()

        plsc.subcore_barrier()
        pltpu.sync_copy(acc.at[pl.ds(s * rpt, rpt)],
                        out_hbm.at[c].at[pl.ds(s * rpt, rpt)])

    return k


GB = 40


def _agg_kernel(n_tot, h, nwin):
    rpt = n_tot // NS
    assert nwin % GB == 0
    nchunk = nwin // GB

    @functools.partial(
        pl.kernel,
        out_type=jax.ShapeDtypeStruct((NC, n_tot, h), jnp.float32),
        mesh=_sc_mesh(),
        scratch_types=[
            pltpu.VMEM((GB, KW), jnp.int32),
            pltpu.VMEM((GB, KW), jnp.int32),
            pltpu.VMEM((KW, h), jnp.float32),
            pltpu.VMEM((KW, h), jnp.float32),
            pltpu.VMEM_SHARED((n_tot, h), jnp.float32),
            pltpu.SemaphoreType.DMA,
            pltpu.SemaphoreType.DMA,
            pltpu.SemaphoreType.DMA,
            pltpu.SemaphoreType.DMA,
        ],
    )
    def k(g_hbm, src_hbm, dst_hbm, zeros_hbm, out_hbm,
          srcv, dstv, b0, b1, acc, gs0, gs1, ss0, ss1):
        c = lax.axis_index("c")
        s = lax.axis_index("s")
        wid = c * NS + s
        bb = (b0, b1)
        gss = (gs0, gs1)
        sss = (ss0, ss1)
        pltpu.sync_copy(zeros_hbm.at[pl.ds(s * rpt, rpt)],
                        acc.at[pl.ds(s * rpt, rpt)])
        plsc.subcore_barrier()

        @pl.loop(0, nchunk)
        def _(ci):
            pltpu.sync_copy(src_hbm.at[wid].at[pl.ds(ci * GB, GB)], srcv)
            pltpu.sync_copy(dst_hbm.at[wid].at[pl.ds(ci * GB, GB)], dstv)
            hg = [None, None]
            hs = [None, None]
            for j in range(GB):
                p = j & 1
                if hs[p] is not None:
                    hs[p].wait()
                hg[p] = pltpu.async_copy(g_hbm.at[srcv.at[j]], bb[p], gss[p])
                if j >= 1:
                    q = 1 - p
                    hg[q].wait()
                    hs[q] = pltpu.async_copy(bb[q], acc.at[dstv.at[j - 1]],
                                             sss[q], add=True)
            p = (GB - 1) & 1
            hg[p].wait()
            hs[p] = pltpu.async_copy(bb[p], acc.at[dstv.at[GB - 1]],
                                     sss[p], add=True)
            hs[0].wait()
            hs[1].wait()

        plsc.subcore_barrier()
        pltpu.sync_copy(acc.at[pl.ds(s * rpt, rpt)],
                        out_hbm.at[c].at[pl.ds(s * rpt, rpt)])

    return k


def _agg1_kernel(n_tot, nwin):
    rpt = n_tot // NS

    cp = pltpu.CompilerParams()
    if "needs_layout_passes" in pltpu.CompilerParams.__dataclass_fields__:
        cp = dataclasses.replace(cp, needs_layout_passes=False)

    @functools.partial(
        pl.kernel,
        out_type=jax.ShapeDtypeStruct((NC, n_tot), jnp.float32),
        mesh=_sc_mesh(),
        compiler_params=cp,
        scratch_types=[
            pltpu.VMEM((n_tot,), jnp.float32),
            pltpu.VMEM((nwin, KW), jnp.int32),
            pltpu.VMEM((nwin, KW), jnp.int32),
            pltpu.VMEM((nwin, KW), jnp.float32),
            pltpu.VMEM_SHARED((n_tot,), jnp.float32),
            pltpu.SemaphoreType.DMA,
        ],
    )
    def k(z_hbm, src_hbm, dst_hbm, zeros_hbm, out_hbm, zv, srcv, dstv, vals,
          acc, ss):
        c = lax.axis_index("c")
        s = lax.axis_index("s")
        wid = c * NS + s
        pltpu.sync_copy(z_hbm, zv)
        pltpu.sync_copy(src_hbm.at[wid], srcv)
        pltpu.sync_copy(dst_hbm.at[wid], dstv)
        pltpu.sync_copy(zeros_hbm.at[pl.ds(s * rpt, rpt)],
                        acc.at[pl.ds(s * rpt, rpt)])
        plsc.subcore_barrier()

        @pl.loop(0, nwin)
        def _(w):
            for g in range(KW // 16):
                idx = srcv[w, pl.ds(16 * g, 16)]
                vals[w, pl.ds(16 * g, 16)] = plsc.load_gather(zv, [idx])

        @pl.loop(0, nwin)
        def _(w):
            pltpu.async_copy(vals.at[w], acc.at[dstv.at[w]], ss, add=True)

        @pl.loop(0, nwin)
        def _(w):
            pltpu.make_async_copy(vals.at[w], acc.at[dstv.at[w]], ss).wait()

        plsc.subcore_barrier()
        pltpu.sync_copy(acc.at[pl.ds(s * rpt, rpt)],
                        out_hbm.at[c].at[pl.ds(s * rpt, rpt)])

    return k


def _scaled_matmul(x, w, deg0, deg1, n_real, bn):
    n, d = x.shape
    h = w.shape[1]

    def body(x_ref, w_ref, d0_ref, d1_ref, g_ref, dinv_ref):
        i = pl.program_id(0)
        rid = i * bn + lax.broadcasted_iota(jnp.int32, (bn,), 0)
        dinv = lax.rsqrt(1.0 + d0_ref[...] + d1_ref[...])
        dinv = jnp.where(rid < n_real, dinv, 0.0)
        dinv_ref[...] = dinv
        g_ref[...] = jnp.dot(x_ref[...] * dinv[:, None], w_ref[...],
                             precision=lax.Precision.HIGHEST,
                             preferred_element_type=jnp.float32)

    return pl.pallas_call(
        body,
        grid=(n // bn,),
        in_specs=[pl.BlockSpec((bn, d), lambda i: (i, 0)),
                  pl.BlockSpec((d, h), lambda i: (0, 0)),
                  pl.BlockSpec((bn,), lambda i: (i,)),
                  pl.BlockSpec((bn,), lambda i: (i,))],
        out_specs=[pl.BlockSpec((bn, h), lambda i: (i, 0)),
                   pl.BlockSpec((bn,), lambda i: (i,))],
        out_shape=[jax.ShapeDtypeStruct((n, h), jnp.float32),
                   jax.ShapeDtypeStruct((n,), jnp.float32)],
    )(x, w, deg0, deg1)


def _layer2_z(s0, s1, g, dinv, b1, w2row, bn):
    n_tot, hdim = g.shape

    def body(s0_ref, s1_ref, g_ref, dinv_ref, b1_ref, w2_ref, o_ref):
        dinv = dinv_ref[...]
        t = dinv[:, None] * (s0_ref[...] + s1_ref[...] + g_ref[...])
        t = jnp.maximum(t + b1_ref[...], 0.0)
        o_ref[...] = jnp.sum(t * w2_ref[...], axis=1) * dinv

    return pl.pallas_call(
        body,
        grid=(n_tot // bn,),
        in_specs=[pl.BlockSpec((bn, hdim), lambda i: (i, 0)),
                  pl.BlockSpec((bn, hdim), lambda i: (i, 0)),
                  pl.BlockSpec((bn, hdim), lambda i: (i, 0)),
                  pl.BlockSpec((bn,), lambda i: (i,)),
                  pl.BlockSpec((1, hdim), lambda i: (0, 0)),
                  pl.BlockSpec((1, hdim), lambda i: (0, 0))],
        out_specs=pl.BlockSpec((bn,), lambda i: (i,)),
        out_shape=jax.ShapeDtypeStruct((n_tot,), jnp.float32),
    )(s0, s1, g, dinv, b1, w2row)


def _final(s0, s1, zp, dinv, b2, bn):
    n = s0.shape[0]

    def body(s0_ref, s1_ref, z_ref, dinv_ref, b2_ref, o_ref):
        o_ref[...] = (dinv_ref[...] * (s0_ref[...] + s1_ref[...] + z_ref[...])
                      + b2_ref[...])

    return pl.pallas_call(
        body,
        grid=(n // bn,),
        in_specs=[pl.BlockSpec((bn,), lambda i: (i,)),
                  pl.BlockSpec((bn,), lambda i: (i,)),
                  pl.BlockSpec((bn,), lambda i: (i,)),
                  pl.BlockSpec((bn,), lambda i: (i,)),
                  pl.BlockSpec((1,), lambda i: (0,))],
        out_specs=pl.BlockSpec((bn,), lambda i: (i,)),
        out_shape=jax.ShapeDtypeStruct((n,), jnp.float32),
    )(s0, s1, zp, dinv, b2)


def kernel(x, edge_index, W1, b1, W2, b2):
    n, d = x.shape
    h = W1.shape[1]
    e = edge_index.shape[1]

    pad_rows = 240
    n_tot = n + pad_rows
    nwin = -(-e // (NW * KW))
    nwin = -(-nwin // GB) * GB
    e_pad = NW * nwin * KW

    npad = e_pad - e
    pad_idx = (n + jnp.arange(npad, dtype=jnp.int32) % pad_rows)
    src3 = jnp.concatenate([edge_index[0].astype(jnp.int32), pad_idx])
    dst3 = jnp.concatenate([edge_index[1].astype(jnp.int32), pad_idx])
    src3 = src3.reshape(NW, nwin, KW)
    dst3 = dst3.reshape(NW, nwin, KW)

    zeros1 = jnp.zeros((n_tot,), jnp.float32)
    zeros2 = jnp.zeros((n_tot, h), jnp.float32)
    ones = jnp.ones((KW,), jnp.float32)

    x_ext = jnp.pad(x, ((0, pad_rows), (0, 0)))

    degp = _deg_kernel(n_tot, nwin)(dst3, zeros1, ones)
    g_ext, dinv = _scaled_matmul(x_ext, W1, degp[0], degp[1], n, bn=1024)

    sp = _agg_kernel(n_tot, h, nwin)(g_ext, src3, dst3, zeros2)

    zp = _layer2_z(sp[0], sp[1], g_ext, dinv, b1.reshape(1, h),
                   W2.reshape(1, h), bn=1024)

    s2p = _agg1_kernel(n_tot, nwin)(zp, src3, dst3, zeros1)

    return _final(s2p[0], s2p[1], zp, dinv, b2, bn=1024)[:n]
